# Initial kernel scaffold; baseline (speedup 1.0000x reference)
import jax
import jax.numpy as jnp
from jax.experimental import pallas as pl


def kernel(edge_index, emb, alpha):
    raise NotImplementedError("write your pallas kernel here")


# speedup vs baseline: 4.9519x; 4.9519x over previous
import functools

import jax
import jax.numpy as jnp
from jax import lax
from jax.experimental import pallas as pl
from jax.experimental.pallas import tpu as pltpu
from jax.experimental.pallas import tpu_sc as plsc

N = 10000
D = 128
E = 320000
LAYERS = 3

NC, NS = 2, 16
NW = NC * NS
BIG = 256
EPT = 10240
GED = 2560
E_PAD = EPT * NW
N_PAD = 10240
RPT = N_PAD // NS
DEGW = 128

_mesh = plsc.VectorSubcoreMesh(
    core_axis_name="c", subcore_axis_name="s", num_cores=NC, num_subcores=NS
)


@functools.partial(
    pl.kernel,
    out_type=jax.ShapeDtypeStruct((NC, N_PAD, DEGW), jnp.float32),
    mesh=_mesh,
    scratch_types=[
        pltpu.VMEM((GED,), jnp.int32),
        pltpu.VMEM((BIG, DEGW), jnp.float32),
        pltpu.VMEM_SHARED((N_PAD, DEGW), jnp.float32),
    ],
)
def _deg_kernel(coli, zdeg, ones_hbm, degp, cidx, ones_v, dacc):
    c = lax.axis_index("c")
    s = lax.axis_index("s")
    w = s * NC + c
    pltpu.sync_copy(zdeg.at[pl.ds(s * RPT, RPT), :], dacc.at[pl.ds(s * RPT, RPT), :])
    pltpu.sync_copy(ones_hbm, ones_v)
    plsc.subcore_barrier()

    def group(g, carry):
        pltpu.sync_copy(coli.at[pl.ds(w * EPT + g * GED, GED)], cidx)

        def body(j, carry2):
            pltpu.sync_copy(ones_v, dacc.at[cidx.at[pl.ds(j * BIG, BIG)]], add=True)
            return carry2

        lax.fori_loop(0, GED // BIG, body, 0)
        return carry

    lax.fori_loop(0, EPT // GED, group, 0)
    plsc.subcore_barrier()
    pltpu.sync_copy(
        dacc.at[pl.ds(s * RPT, RPT), :], degp.at[c, pl.ds(s * RPT, RPT), :]
    )


@functools.partial(
    pl.kernel,
    out_type=jax.ShapeDtypeStruct((NC, N_PAD, D), jnp.float32),
    mesh=_mesh,
    scratch_types=[
        pltpu.VMEM((GED,), jnp.int32),
        pltpu.VMEM((GED,), jnp.int32),
        pltpu.VMEM((BIG, D), jnp.float32),
        pltpu.VMEM_SHARED((N_PAD, D), jnp.float32),
        pltpu.SemaphoreType.DMA,
    ],
)
def _scatter_kernel(z_hbm, rowi, coli, zrows, part, ridx, cidx, gbuf, acc, sem):
    c = lax.axis_index("c")
    s = lax.axis_index("s")
    w = s * NC + c
    pltpu.sync_copy(zrows.at[pl.ds(s * RPT, RPT), :], acc.at[pl.ds(s * RPT, RPT), :])
    plsc.subcore_barrier()

    def group(g, carry):
        base = w * EPT + g * GED
        pltpu.sync_copy(rowi.at[pl.ds(base, GED)], ridx)
        pltpu.sync_copy(coli.at[pl.ds(base, GED)], cidx)

        def body(j, carry2):
            cp = pltpu.async_copy(z_hbm.at[ridx.at[pl.ds(j * BIG, BIG)]], gbuf, sem)
            cp.wait()
            pltpu.sync_copy(gbuf, acc.at[cidx.at[pl.ds(j * BIG, BIG)]], add=True)
            return carry2

        lax.fori_loop(0, GED // BIG, body, 0)
        return carry

    lax.fori_loop(0, EPT // GED, group, 0)
    plsc.subcore_barrier()
    pltpu.sync_copy(
        acc.at[pl.ds(s * RPT, RPT), :], part.at[c, pl.ds(s * RPT, RPT), :]
    )


RB = 1280
GP = N_PAD // RB


def _prep_body(alpha_ref, degp_ref, emb_ref, dinv_ref, dinv2_ref, z0_ref, oacc_ref):
    deg = degp_ref[0, :, 0:1] + degp_ref[1, :, 0:1]
    pos = deg > 0.0
    dinv = jnp.where(pos, lax.rsqrt(jnp.where(pos, deg, 1.0)), 0.0)
    dinv_ref[...] = dinv
    dinv2_ref[...] = dinv * dinv
    e = emb_ref[...]
    z0_ref[...] = e * dinv
    oacc_ref[...] = e * alpha_ref[0]


_prep = pl.pallas_call(
    _prep_body,
    grid=(GP,),
    in_specs=[
        pl.BlockSpec(memory_space=pltpu.SMEM),
        pl.BlockSpec((2, RB, DEGW), lambda i: (0, i, 0)),
        pl.BlockSpec((RB, D), lambda i: (i, 0)),
    ],
    out_specs=[
        pl.BlockSpec((RB, 1), lambda i: (i, 0)),
        pl.BlockSpec((RB, 1), lambda i: (i, 0)),
        pl.BlockSpec((RB, D), lambda i: (i, 0)),
        pl.BlockSpec((RB, D), lambda i: (i, 0)),
    ],
    out_shape=[
        jax.ShapeDtypeStruct((N_PAD, 1), jnp.float32),
        jax.ShapeDtypeStruct((N_PAD, 1), jnp.float32),
        jax.ShapeDtypeStruct((N_PAD, D), jnp.float32),
        jax.ShapeDtypeStruct((N_PAD, D), jnp.float32),
    ],
)


def _comb_body(l, alpha_ref, part_ref, dinv_ref, dinv2_ref, oin_ref, z_ref, oout_ref):
    sm = part_ref[0] + part_ref[1]
    z_ref[...] = sm * dinv2_ref[...]
    oout_ref[...] = oin_ref[...] + (sm * dinv_ref[...]) * alpha_ref[l]


def _make_comb(l):
    return pl.pallas_call(
        functools.partial(_comb_body, l),
        grid=(GP,),
        in_specs=[
            pl.BlockSpec(memory_space=pltpu.SMEM),
            pl.BlockSpec((2, RB, D), lambda i: (0, i, 0)),
            pl.BlockSpec((RB, 1), lambda i: (i, 0)),
            pl.BlockSpec((RB, 1), lambda i: (i, 0)),
            pl.BlockSpec((RB, D), lambda i: (i, 0)),
        ],
        out_specs=[
            pl.BlockSpec((RB, D), lambda i: (i, 0)),
            pl.BlockSpec((RB, D), lambda i: (i, 0)),
        ],
        out_shape=[
            jax.ShapeDtypeStruct((N_PAD, D), jnp.float32),
            jax.ShapeDtypeStruct((N_PAD, D), jnp.float32),
        ],
    )


_combs = [None] + [_make_comb(l) for l in range(1, LAYERS + 1)]


def kernel(edge_index, emb, alpha):
    row = edge_index[0].astype(jnp.int32)
    col = edge_index[1].astype(jnp.int32)
    pad = jnp.full((E_PAD - E,), N, jnp.int32)
    rowi = jnp.concatenate([row, pad])
    coli = jnp.concatenate([col, pad])
    emb_pad = jnp.zeros((N_PAD, D), jnp.float32).at[:N].set(emb)
    zdeg = jnp.zeros((N_PAD, DEGW), jnp.float32)
    ones = jnp.ones((BIG, DEGW), jnp.float32)
    zrows = jnp.zeros((N_PAD, D), jnp.float32)

    degp = _deg_kernel(coli, zdeg, ones)
    dinv, dinv2, z, oacc = _prep(alpha, degp, emb_pad)
    for l in range(1, LAYERS + 1):
        part = _scatter_kernel(z, rowi, coli, zrows)
        z, oacc = _combs[l](alpha, part, dinv, dinv2, oacc)
    return oacc[:N]


# speedup vs baseline: 5.3056x; 1.0714x over previous
import functools

import jax
import jax.numpy as jnp
from jax import lax
from jax.experimental import pallas as pl
from jax.experimental.pallas import tpu as pltpu
from jax.experimental.pallas import tpu_sc as plsc

N = 10000
D = 128
E = 320000
LAYERS = 3

NC, NS = 2, 16
NW = NC * NS
BIG = 256
CH = 160
EPT = 10240
GED = 2560
E_PAD = EPT * NW
N_PAD = 10240
RPT = N_PAD // NS
DEGW = 128

_mesh = plsc.VectorSubcoreMesh(
    core_axis_name="c", subcore_axis_name="s", num_cores=NC, num_subcores=NS
)


@functools.partial(
    pl.kernel,
    out_type=jax.ShapeDtypeStruct((NC, N_PAD, DEGW), jnp.float32),
    mesh=_mesh,
    scratch_types=[
        pltpu.VMEM((GED,), jnp.int32),
        pltpu.VMEM((BIG, DEGW), jnp.float32),
        pltpu.VMEM_SHARED((N_PAD, DEGW), jnp.float32),
    ],
)
def _deg_kernel(coli, zdeg, ones_hbm, degp, cidx, ones_v, dacc):
    c = lax.axis_index("c")
    s = lax.axis_index("s")
    w = s * NC + c
    pltpu.sync_copy(zdeg.at[pl.ds(s * RPT, RPT), :], dacc.at[pl.ds(s * RPT, RPT), :])
    pltpu.sync_copy(ones_hbm, ones_v)
    plsc.subcore_barrier()

    def group(g, carry):
        pltpu.sync_copy(coli.at[pl.ds(w * EPT + g * GED, GED)], cidx)

        def body(j, carry2):
            pltpu.sync_copy(ones_v, dacc.at[cidx.at[pl.ds(j * BIG, BIG)]], add=True)
            return carry2

        lax.fori_loop(0, GED // BIG, body, 0)
        return carry

    lax.fori_loop(0, EPT // GED, group, 0)
    plsc.subcore_barrier()
    pltpu.sync_copy(
        dacc.at[pl.ds(s * RPT, RPT), :], degp.at[c, pl.ds(s * RPT, RPT), :]
    )


@functools.partial(
    pl.kernel,
    out_type=jax.ShapeDtypeStruct((NC, N_PAD, D), jnp.float32),
    mesh=_mesh,
    scratch_types=[
        pltpu.VMEM((GED,), jnp.int32),
        pltpu.VMEM((GED,), jnp.int32),
        pltpu.VMEM((CH, D), jnp.float32),
        pltpu.VMEM((CH, D), jnp.float32),
        pltpu.VMEM_SHARED((N_PAD, D), jnp.float32),
        pltpu.SemaphoreType.DMA,
        pltpu.SemaphoreType.DMA,
    ],
)
def _scatter_kernel(z_hbm, rowi, coli, zrows, part, ridx, cidx, gbuf0, gbuf1,
                    acc, sem0, sem1):
    c = lax.axis_index("c")
    s = lax.axis_index("s")
    w = s * NC + c
    pltpu.sync_copy(zrows.at[pl.ds(s * RPT, RPT), :], acc.at[pl.ds(s * RPT, RPT), :])
    plsc.subcore_barrier()

    NCH = GED // CH

    def group(g, carry):
        base = w * EPT + g * GED
        pltpu.sync_copy(rowi.at[pl.ds(base, GED)], ridx)
        pltpu.sync_copy(coli.at[pl.ds(base, GED)], cidx)
        pltpu.async_copy(z_hbm.at[ridx.at[pl.ds(0, CH)]], gbuf0, sem0)

        def body(i, carry2):
            j = 2 * i
            pltpu.async_copy(z_hbm.at[ridx.at[pl.ds((j + 1) * CH, CH)]], gbuf1, sem1)
            pltpu.make_async_copy(
                z_hbm.at[ridx.at[pl.ds(j * CH, CH)]], gbuf0, sem0).wait()
            pltpu.sync_copy(gbuf0, acc.at[cidx.at[pl.ds(j * CH, CH)]], add=True)

            @pl.when(j + 2 < NCH)
            def _():
                pltpu.async_copy(
                    z_hbm.at[ridx.at[pl.ds((j + 2) * CH, CH)]], gbuf0, sem0)

            pltpu.make_async_copy(
                z_hbm.at[ridx.at[pl.ds((j + 1) * CH, CH)]], gbuf1, sem1).wait()
            pltpu.sync_copy(gbuf1, acc.at[cidx.at[pl.ds((j + 1) * CH, CH)]], add=True)
            return carry2

        lax.fori_loop(0, NCH // 2, body, 0)
        return carry

    lax.fori_loop(0, EPT // GED, group, 0)
    plsc.subcore_barrier()
    pltpu.sync_copy(
        acc.at[pl.ds(s * RPT, RPT), :], part.at[c, pl.ds(s * RPT, RPT), :]
    )


RB = 1280
GP = N_PAD // RB


def _prep_body(alpha_ref, degp_ref, emb_ref, dinv_ref, dinv2_ref, z0_ref, oacc_ref):
    deg = degp_ref[0, :, 0:1] + degp_ref[1, :, 0:1]
    pos = deg > 0.0
    dinv = jnp.where(pos, lax.rsqrt(jnp.where(pos, deg, 1.0)), 0.0)
    dinv_ref[...] = dinv
    dinv2_ref[...] = dinv * dinv
    e = emb_ref[...]
    z0_ref[...] = e * dinv
    oacc_ref[...] = e * alpha_ref[0]


_prep = pl.pallas_call(
    _prep_body,
    grid=(GP,),
    in_specs=[
        pl.BlockSpec(memory_space=pltpu.SMEM),
        pl.BlockSpec((2, RB, DEGW), lambda i: (0, i, 0)),
        pl.BlockSpec((RB, D), lambda i: (i, 0)),
    ],
    out_specs=[
        pl.BlockSpec((RB, 1), lambda i: (i, 0)),
        pl.BlockSpec((RB, 1), lambda i: (i, 0)),
        pl.BlockSpec((RB, D), lambda i: (i, 0)),
        pl.BlockSpec((RB, D), lambda i: (i, 0)),
    ],
    out_shape=[
        jax.ShapeDtypeStruct((N_PAD, 1), jnp.float32),
        jax.ShapeDtypeStruct((N_PAD, 1), jnp.float32),
        jax.ShapeDtypeStruct((N_PAD, D), jnp.float32),
        jax.ShapeDtypeStruct((N_PAD, D), jnp.float32),
    ],
)


def _comb_body(l, alpha_ref, part_ref, dinv_ref, dinv2_ref, oin_ref, z_ref, oout_ref):
    sm = part_ref[0] + part_ref[1]
    z_ref[...] = sm * dinv2_ref[...]
    oout_ref[...] = oin_ref[...] + (sm * dinv_ref[...]) * alpha_ref[l]


def _make_comb(l):
    return pl.pallas_call(
        functools.partial(_comb_body, l),
        grid=(GP,),
        in_specs=[
            pl.BlockSpec(memory_space=pltpu.SMEM),
            pl.BlockSpec((2, RB, D), lambda i: (0, i, 0)),
            pl.BlockSpec((RB, 1), lambda i: (i, 0)),
            pl.BlockSpec((RB, 1), lambda i: (i, 0)),
            pl.BlockSpec((RB, D), lambda i: (i, 0)),
        ],
        out_specs=[
            pl.BlockSpec((RB, D), lambda i: (i, 0)),
            pl.BlockSpec((RB, D), lambda i: (i, 0)),
        ],
        out_shape=[
            jax.ShapeDtypeStruct((N_PAD, D), jnp.float32),
            jax.ShapeDtypeStruct((N_PAD, D), jnp.float32),
        ],
    )


_combs = [None] + [_make_comb(l) for l in range(1, LAYERS + 1)]


def kernel(edge_index, emb, alpha):
    row = edge_index[0].astype(jnp.int32)
    col = edge_index[1].astype(jnp.int32)
    pad = jnp.full((E_PAD - E,), N, jnp.int32)
    rowi = jnp.concatenate([row, pad])
    coli = jnp.concatenate([col, pad])
    emb_pad = jnp.zeros((N_PAD, D), jnp.float32).at[:N].set(emb)
    zdeg = jnp.zeros((N_PAD, DEGW), jnp.float32)
    ones = jnp.ones((BIG, DEGW), jnp.float32)
    zrows = jnp.zeros((N_PAD, D), jnp.float32)

    degp = _deg_kernel(coli, zdeg, ones)
    dinv, dinv2, z, oacc = _prep(alpha, degp, emb_pad)
    for l in range(1, LAYERS + 1):
        part = _scatter_kernel(z, rowi, coli, zrows)
        z, oacc = _combs[l](alpha, part, dinv, dinv2, oacc)
    return oacc[:N]


# speedup vs baseline: 6.0863x; 1.1471x over previous
import functools

import jax
import jax.numpy as jnp
from jax import lax
from jax.experimental import pallas as pl
from jax.experimental.pallas import tpu as pltpu
from jax.experimental.pallas import tpu_sc as plsc

N = 10000
D = 128
E = 320000
LAYERS = 3

NC, NS = 2, 16
NW = NC * NS
CHUNK = 128
KCH = 80
GCH = 40
EPT = CHUNK * KCH
E_PAD = EPT * NW
N_PAD = 10240
RPT = N_PAD // NS
DEGW = 128

_mesh = plsc.VectorSubcoreMesh(
    core_axis_name="c", subcore_axis_name="s", num_cores=NC, num_subcores=NS
)


@functools.partial(
    pl.kernel,
    out_type=jax.ShapeDtypeStruct((NC, N_PAD, DEGW), jnp.float32),
    mesh=_mesh,
    scratch_types=[
        pltpu.VMEM((KCH, CHUNK), jnp.int32),
        pltpu.VMEM((CHUNK, DEGW), jnp.float32),
        pltpu.VMEM_SHARED((N_PAD, DEGW), jnp.float32),
    ],
)
def _deg_kernel(coli, zdeg, ones_hbm, degp, cidx, ones_v, dacc):
    c = lax.axis_index("c")
    s = lax.axis_index("s")
    w = s * NC + c
    pltpu.sync_copy(zdeg.at[pl.ds(s * RPT, RPT), :], dacc.at[pl.ds(s * RPT, RPT), :])
    pltpu.sync_copy(coli.at[pl.ds(w * KCH, KCH), :], cidx)
    pltpu.sync_copy(ones_hbm, ones_v)
    plsc.subcore_barrier()

    def body(j, carry):
        pltpu.sync_copy(ones_v, dacc.at[cidx.at[j]], add=True)
        return carry

    lax.fori_loop(0, KCH, body, 0)
    plsc.subcore_barrier()
    pltpu.sync_copy(
        dacc.at[pl.ds(s * RPT, RPT), :], degp.at[c, pl.ds(s * RPT, RPT), :]
    )


@functools.partial(
    pl.kernel,
    out_type=jax.ShapeDtypeStruct((NC, N_PAD, D), jnp.float32),
    mesh=_mesh,
    scratch_types=[
        pltpu.VMEM((GCH, CHUNK), jnp.int32),
        pltpu.VMEM((GCH, CHUNK), jnp.int32),
        pltpu.VMEM((CHUNK, D), jnp.float32),
        pltpu.VMEM((CHUNK, D), jnp.float32),
        pltpu.VMEM_SHARED((N_PAD, D), jnp.float32),
        pltpu.SemaphoreType.DMA,
        pltpu.SemaphoreType.DMA,
    ],
)
def _scatter_kernel(z_hbm, rowi, coli, zrows, part, ridx, cidx, gbuf0, gbuf1,
                    acc, sem0, sem1):
    c = lax.axis_index("c")
    s = lax.axis_index("s")
    w = s * NC + c
    pltpu.sync_copy(zrows.at[pl.ds(s * RPT, RPT), :], acc.at[pl.ds(s * RPT, RPT), :])
    plsc.subcore_barrier()

    def group(g, carry):
        pltpu.sync_copy(rowi.at[pl.ds(w * KCH + g * GCH, GCH), :], ridx)
        pltpu.sync_copy(coli.at[pl.ds(w * KCH + g * GCH, GCH), :], cidx)
        pltpu.async_copy(z_hbm.at[ridx.at[0]], gbuf0, sem0)

        def body(i, carry2):
            j = 2 * i
            pltpu.async_copy(z_hbm.at[ridx.at[j + 1]], gbuf1, sem1)
            pltpu.make_async_copy(z_hbm.at[ridx.at[j]], gbuf0, sem0).wait()
            pltpu.sync_copy(gbuf0, acc.at[cidx.at[j]], add=True)

            @pl.when(j + 2 < GCH)
            def _():
                pltpu.async_copy(z_hbm.at[ridx.at[j + 2]], gbuf0, sem0)

            pltpu.make_async_copy(z_hbm.at[ridx.at[j + 1]], gbuf1, sem1).wait()
            pltpu.sync_copy(gbuf1, acc.at[cidx.at[j + 1]], add=True)
            return carry2

        lax.fori_loop(0, GCH // 2, body, 0)
        return carry

    lax.fori_loop(0, KCH // GCH, group, 0)
    plsc.subcore_barrier()
    pltpu.sync_copy(
        acc.at[pl.ds(s * RPT, RPT), :], part.at[c, pl.ds(s * RPT, RPT), :]
    )


RB = 1280
GP = N_PAD // RB


def _prep_body(alpha_ref, degp_ref, emb_ref, dinv_ref, dinv2_ref, z0_ref, oacc_ref):
    deg = degp_ref[0, :, 0:1] + degp_ref[1, :, 0:1]
    pos = deg > 0.0
    dinv = jnp.where(pos, lax.rsqrt(jnp.where(pos, deg, 1.0)), 0.0)
    dinv_ref[...] = dinv
    dinv2_ref[...] = dinv * dinv
    e = emb_ref[...]
    z0_ref[...] = e * dinv
    oacc_ref[...] = e * alpha_ref[0]


_prep = pl.pallas_call(
    _prep_body,
    grid=(GP,),
    in_specs=[
        pl.BlockSpec(memory_space=pltpu.SMEM),
        pl.BlockSpec((2, RB, DEGW), lambda i: (0, i, 0)),
        pl.BlockSpec((RB, D), lambda i: (i, 0)),
    ],
    out_specs=[
        pl.BlockSpec((RB, 1), lambda i: (i, 0)),
        pl.BlockSpec((RB, 1), lambda i: (i, 0)),
        pl.BlockSpec((RB, D), lambda i: (i, 0)),
        pl.BlockSpec((RB, D), lambda i: (i, 0)),
    ],
    out_shape=[
        jax.ShapeDtypeStruct((N_PAD, 1), jnp.float32),
        jax.ShapeDtypeStruct((N_PAD, 1), jnp.float32),
        jax.ShapeDtypeStruct((N_PAD, D), jnp.float32),
        jax.ShapeDtypeStruct((N_PAD, D), jnp.float32),
    ],
)


def _comb_body(l, alpha_ref, part_ref, dinv_ref, dinv2_ref, oin_ref, z_ref, oout_ref):
    sm = part_ref[0] + part_ref[1]
    z_ref[...] = sm * dinv2_ref[...]
    oout_ref[...] = oin_ref[...] + (sm * dinv_ref[...]) * alpha_ref[l]


def _make_comb(l):
    return pl.pallas_call(
        functools.partial(_comb_body, l),
        grid=(GP,),
        in_specs=[
            pl.BlockSpec(memory_space=pltpu.SMEM),
            pl.BlockSpec((2, RB, D), lambda i: (0, i, 0)),
            pl.BlockSpec((RB, 1), lambda i: (i, 0)),
            pl.BlockSpec((RB, 1), lambda i: (i, 0)),
            pl.BlockSpec((RB, D), lambda i: (i, 0)),
        ],
        out_specs=[
            pl.BlockSpec((RB, D), lambda i: (i, 0)),
            pl.BlockSpec((RB, D), lambda i: (i, 0)),
        ],
        out_shape=[
            jax.ShapeDtypeStruct((N_PAD, D), jnp.float32),
            jax.ShapeDtypeStruct((N_PAD, D), jnp.float32),
        ],
    )


_combs = [None] + [_make_comb(l) for l in range(1, LAYERS + 1)]


def kernel(edge_index, emb, alpha):
    row = edge_index[0].astype(jnp.int32)
    col = edge_index[1].astype(jnp.int32)
    pad = jnp.full((E_PAD - E,), N, jnp.int32)
    rowi = jnp.concatenate([row, pad]).reshape(NW * KCH, CHUNK)
    coli = jnp.concatenate([col, pad]).reshape(NW * KCH, CHUNK)
    emb_pad = jnp.zeros((N_PAD, D), jnp.float32).at[:N].set(emb)
    zdeg = jnp.zeros((N_PAD, DEGW), jnp.float32)
    ones = jnp.ones((CHUNK, DEGW), jnp.float32)
    zrows = jnp.zeros((N_PAD, D), jnp.float32)

    degp = _deg_kernel(coli, zdeg, ones)
    dinv, dinv2, z, oacc = _prep(alpha, degp, emb_pad)
    for l in range(1, LAYERS + 1):
        part = _scatter_kernel(z, rowi, coli, zrows)
        z, oacc = _combs[l](alpha, part, dinv, dinv2, oacc)
    return oacc[:N]


# speedup vs baseline: 6.3224x; 1.0388x over previous
import functools

import jax
import jax.numpy as jnp
from jax import lax
from jax.experimental import pallas as pl
from jax.experimental.pallas import tpu as pltpu
from jax.experimental.pallas import tpu_sc as plsc

N = 10000
D = 128
E = 320000
LAYERS = 3

NC, NS = 2, 16
NW = NC * NS
CHUNK = 128
KCH0 = 120
KCH1 = 40
KCHMAX = KCH0
KCHD = (KCH0 + KCH1) // NC
GCH = 40
E_PAD = CHUNK * NS * (KCH0 + KCH1)
N_PAD = 10240
RPT = N_PAD // NS
DEGW = 128

_mesh = plsc.VectorSubcoreMesh(
    core_axis_name="c", subcore_axis_name="s", num_cores=NC, num_subcores=NS
)


@functools.partial(
    pl.kernel,
    out_type=jax.ShapeDtypeStruct((NC, N_PAD, DEGW), jnp.float32),
    mesh=_mesh,
    scratch_types=[
        pltpu.VMEM((KCHD, CHUNK), jnp.int32),
        pltpu.VMEM((CHUNK, DEGW), jnp.float32),
        pltpu.VMEM_SHARED((N_PAD, DEGW), jnp.float32),
    ],
)
def _deg_kernel(coli, zdeg, ones_hbm, degp, cidx, ones_v, dacc):
    c = lax.axis_index("c")
    s = lax.axis_index("s")
    w = s * NC + c
    pltpu.sync_copy(zdeg.at[pl.ds(s * RPT, RPT), :], dacc.at[pl.ds(s * RPT, RPT), :])
    pltpu.sync_copy(coli.at[pl.ds(w * KCHD, KCHD), :], cidx)
    pltpu.sync_copy(ones_hbm, ones_v)
    plsc.subcore_barrier()

    def body(j, carry):
        pltpu.sync_copy(ones_v, dacc.at[cidx.at[j]], add=True)
        return carry

    lax.fori_loop(0, KCHD, body, 0)
    plsc.subcore_barrier()
    pltpu.sync_copy(
        dacc.at[pl.ds(s * RPT, RPT), :], degp.at[c, pl.ds(s * RPT, RPT), :]
    )


@functools.partial(
    pl.kernel,
    out_type=jax.ShapeDtypeStruct((NC, N_PAD, D), jnp.float32),
    mesh=_mesh,
    scratch_types=[
        pltpu.VMEM((GCH, CHUNK), jnp.int32),
        pltpu.VMEM((GCH, CHUNK), jnp.int32),
        pltpu.VMEM((CHUNK, D), jnp.float32),
        pltpu.VMEM((CHUNK, D), jnp.float32),
        pltpu.VMEM_SHARED((N_PAD, D), jnp.float32),
        pltpu.SemaphoreType.DMA,
        pltpu.SemaphoreType.DMA,
    ],
)
def _scatter_kernel(z_hbm, rowi, coli, zrows, part, ridx, cidx, gbuf0, gbuf1,
                    acc, sem0, sem1):
    c = lax.axis_index("c")
    s = lax.axis_index("s")
    pltpu.sync_copy(zrows.at[pl.ds(s * RPT, RPT), :], acc.at[pl.ds(s * RPT, RPT), :])
    plsc.subcore_barrier()

    kch = jnp.where(c == 0, KCH0, KCH1)
    wbase = jnp.where(c == 0, s * KCH0, NS * KCH0 + s * KCH1)

    for g in range(KCHMAX // GCH):

        @pl.when(g * GCH < kch)
        def _():
            pltpu.sync_copy(rowi.at[pl.ds(wbase + g * GCH, GCH), :], ridx)
            pltpu.sync_copy(coli.at[pl.ds(wbase + g * GCH, GCH), :], cidx)
            pltpu.async_copy(z_hbm.at[ridx.at[0]], gbuf0, sem0)

            def body(i, carry2):
                j = 2 * i
                pltpu.async_copy(z_hbm.at[ridx.at[j + 1]], gbuf1, sem1)
                pltpu.make_async_copy(z_hbm.at[ridx.at[j]], gbuf0, sem0).wait()
                pltpu.sync_copy(gbuf0, acc.at[cidx.at[j]], add=True)

                @pl.when(j + 2 < GCH)
                def _():
                    pltpu.async_copy(z_hbm.at[ridx.at[j + 2]], gbuf0, sem0)

                pltpu.make_async_copy(z_hbm.at[ridx.at[j + 1]], gbuf1, sem1).wait()
                pltpu.sync_copy(gbuf1, acc.at[cidx.at[j + 1]], add=True)
                return carry2

            lax.fori_loop(0, GCH // 2, body, 0)
    plsc.subcore_barrier()
    pltpu.sync_copy(
        acc.at[pl.ds(s * RPT, RPT), :], part.at[c, pl.ds(s * RPT, RPT), :]
    )


RB = 1280
GP = N_PAD // RB


def _prep_body(alpha_ref, degp_ref, emb_ref, dinv_ref, dinv2_ref, z0_ref, oacc_ref):
    deg = degp_ref[0, :, 0:1] + degp_ref[1, :, 0:1]
    pos = deg > 0.0
    dinv = jnp.where(pos, lax.rsqrt(jnp.where(pos, deg, 1.0)), 0.0)
    dinv_ref[...] = dinv
    dinv2_ref[...] = dinv * dinv
    e = emb_ref[...]
    z0_ref[...] = e * dinv
    oacc_ref[...] = e * alpha_ref[0]


_prep = pl.pallas_call(
    _prep_body,
    grid=(GP,),
    in_specs=[
        pl.BlockSpec(memory_space=pltpu.SMEM),
        pl.BlockSpec((2, RB, DEGW), lambda i: (0, i, 0)),
        pl.BlockSpec((RB, D), lambda i: (i, 0)),
    ],
    out_specs=[
        pl.BlockSpec((RB, 1), lambda i: (i, 0)),
        pl.BlockSpec((RB, 1), lambda i: (i, 0)),
        pl.BlockSpec((RB, D), lambda i: (i, 0)),
        pl.BlockSpec((RB, D), lambda i: (i, 0)),
    ],
    out_shape=[
        jax.ShapeDtypeStruct((N_PAD, 1), jnp.float32),
        jax.ShapeDtypeStruct((N_PAD, 1), jnp.float32),
        jax.ShapeDtypeStruct((N_PAD, D), jnp.float32),
        jax.ShapeDtypeStruct((N_PAD, D), jnp.float32),
    ],
)


def _comb_body(l, alpha_ref, part_ref, dinv_ref, dinv2_ref, oin_ref, z_ref, oout_ref):
    sm = part_ref[0] + part_ref[1]
    z_ref[...] = sm * dinv2_ref[...]
    oout_ref[...] = oin_ref[...] + (sm * dinv_ref[...]) * alpha_ref[l]


def _make_comb(l):
    return pl.pallas_call(
        functools.partial(_comb_body, l),
        grid=(GP,),
        in_specs=[
            pl.BlockSpec(memory_space=pltpu.SMEM),
            pl.BlockSpec((2, RB, D), lambda i: (0, i, 0)),
            pl.BlockSpec((RB, 1), lambda i: (i, 0)),
            pl.BlockSpec((RB, 1), lambda i: (i, 0)),
            pl.BlockSpec((RB, D), lambda i: (i, 0)),
        ],
        out_specs=[
            pl.BlockSpec((RB, D), lambda i: (i, 0)),
            pl.BlockSpec((RB, D), lambda i: (i, 0)),
        ],
        out_shape=[
            jax.ShapeDtypeStruct((N_PAD, D), jnp.float32),
            jax.ShapeDtypeStruct((N_PAD, D), jnp.float32),
        ],
    )


_combs = [None] + [_make_comb(l) for l in range(1, LAYERS + 1)]


def kernel(edge_index, emb, alpha):
    row = edge_index[0].astype(jnp.int32)
    col = edge_index[1].astype(jnp.int32)
    pad = jnp.full((E_PAD - E,), N, jnp.int32)
    rowi = jnp.concatenate([row, pad]).reshape(E_PAD // CHUNK, CHUNK)
    coli = jnp.concatenate([col, pad]).reshape(E_PAD // CHUNK, CHUNK)
    emb_pad = jnp.zeros((N_PAD, D), jnp.float32).at[:N].set(emb)
    zdeg = jnp.zeros((N_PAD, DEGW), jnp.float32)
    ones = jnp.ones((CHUNK, DEGW), jnp.float32)
    zrows = jnp.zeros((N_PAD, D), jnp.float32)

    degp = _deg_kernel(coli, zdeg, ones)
    dinv, dinv2, z, oacc = _prep(alpha, degp, emb_pad)
    for l in range(1, LAYERS + 1):
        part = _scatter_kernel(z, rowi, coli, zrows)
        z, oacc = _combs[l](alpha, part, dinv, dinv2, oacc)
    return oacc[:N]
